# Initial kernel scaffold; baseline (speedup 1.0000x reference)
import jax
import jax.numpy as jnp
from jax.experimental import pallas as pl


def kernel(x, edge_index, W1, b1, W2, b2):
    raise NotImplementedError("write your pallas kernel here")


# speedup vs baseline: 36.9606x; 36.9606x over previous
import functools

import jax
import jax.numpy as jnp
from jax import lax
from jax.experimental import pallas as pl
from jax.experimental.pallas import tpu as pltpu
from jax.experimental.pallas import tpu_sc as plsc

NC = 2
NS = 16
NW = NC * NS
CH = 128
KJ = 8
BN = 1000


def _sc_mesh():
    return plsc.VectorSubcoreMesh(core_axis_name="c", subcore_axis_name="s",
                                  num_cores=NC, num_subcores=NS)


def _make_deg_kernel(Ep, Np):
    CPT = Ep // CH // NW
    SUP = CPT // KJ
    SP = Np // NS

    @functools.partial(
        pl.kernel,
        out_type=[jax.ShapeDtypeStruct((Np,), jnp.float32)] * NC,
        mesh=_sc_mesh(),
        scratch_types=[
            pltpu.VMEM((KJ, CH), jnp.int32),
            pltpu.VMEM((CH,), jnp.float32),
            pltpu.VMEM((SP,), jnp.float32),
            pltpu.VMEM_SHARED((Np,), jnp.float32),
        ],
        compiler_params=pltpu.CompilerParams(use_tc_tiling_on_sc=False),
        name="gcn_deg",
    )
    def degk(dst_hbm, out0_hbm, out1_hbm, didx, ones_v, stage, acc):
        c = lax.axis_index("c")
        s = lax.axis_index("s")
        wid = c * NS + s
        base = wid * CPT
        for i in range(CH // 16):
            ones_v[pl.ds(i * 16, 16)] = jnp.ones((16,), jnp.float32)

        def zbody(i, carry):
            stage[pl.ds(i * 16, 16)] = jnp.zeros((16,), jnp.float32)
            return carry

        lax.fori_loop(0, SP // 16, zbody, 0)
        pltpu.sync_copy(stage, acc.at[pl.ds(s * SP, SP)])
        plsc.subcore_barrier()

        def body(sup, carry):
            r0 = base + sup * KJ
            pltpu.sync_copy(dst_hbm.at[pl.ds(r0, KJ)], didx)
            for j in range(KJ):
                pltpu.sync_copy(ones_v, acc.at[didx.at[j]], add=True)
            return carry

        lax.fori_loop(0, SUP, body, 0)
        plsc.subcore_barrier()
        pltpu.sync_copy(acc.at[pl.ds(s * SP, SP)], stage)

        @pl.when(c == 0)
        def _():
            pltpu.sync_copy(stage, out0_hbm.at[pl.ds(s * SP, SP)])

        @pl.when(c == 1)
        def _():
            pltpu.sync_copy(stage, out1_hbm.at[pl.ds(s * SP, SP)])

    return degk


def _make_scatter_kernel(D, Ep, Np, name):
    CPT = Ep // CH // NW
    SUP = CPT // KJ
    SP = Np // NS

    SPH = SP // 16

    @functools.partial(
        pl.kernel,
        out_type=[jax.ShapeDtypeStruct((Np, D), jnp.float32)] * NC,
        mesh=_sc_mesh(),
        scratch_types=[
            pltpu.VMEM((KJ, CH), jnp.int32),
            pltpu.VMEM((KJ, CH), jnp.int32),
            pltpu.VMEM((KJ, CH, D), jnp.float32),
            pltpu.VMEM((SPH, D), jnp.float32),
            pltpu.VMEM_SHARED((Np, D), jnp.float32),
            pltpu.SemaphoreType.DMA,
        ],
        compiler_params=pltpu.CompilerParams(use_tc_tiling_on_sc=False),
        name=name,
    )
    def scat(src_hbm, dst_hbm, tbl_hbm, out0_hbm, out1_hbm,
             sidx, didx, rows, stage, acc, sem):
        c = lax.axis_index("c")
        s = lax.axis_index("s")
        wid = c * NS + s
        base = wid * CPT

        def zbody(i, carry):
            stage[i] = jnp.zeros((D,), jnp.float32)
            return carry

        lax.fori_loop(0, SPH, zbody, 0)

        def ibody(p, carry):
            pltpu.sync_copy(stage, acc.at[pl.ds(s * SP + p * SPH, SPH)])
            return carry

        lax.fori_loop(0, 16, ibody, 0)
        plsc.subcore_barrier()

        def body(sup, carry):
            r0 = base + sup * KJ
            pltpu.sync_copy(src_hbm.at[pl.ds(r0, KJ)], sidx)
            pltpu.sync_copy(dst_hbm.at[pl.ds(r0, KJ)], didx)
            cps = [pltpu.async_copy(tbl_hbm.at[sidx.at[j]], rows.at[j], sem)
                   for j in range(KJ)]
            for cp in cps:
                cp.wait()
            for j in range(KJ):
                pltpu.sync_copy(rows.at[j], acc.at[didx.at[j]], add=True)
            return carry

        lax.fori_loop(0, SUP, body, 0)
        plsc.subcore_barrier()

        def obody(p, carry):
            sl = pl.ds(s * SP + p * SPH, SPH)
            pltpu.sync_copy(acc.at[sl], stage)

            @pl.when(c == 0)
            def _():
                pltpu.sync_copy(stage, out0_hbm.at[sl])

            @pl.when(c == 1)
            def _():
                pltpu.sync_copy(stage, out1_hbm.at[sl])
            return carry

        lax.fori_loop(0, 16, obody, 0)

    return scat


def _mm_body(x_ref, w_ref, d0_ref, d1_ref, hs_ref, dinv_ref):
    h = jnp.dot(x_ref[...], w_ref[...], preferred_element_type=jnp.float32)
    deg = d0_ref[...] + d1_ref[...] + 1.0
    dinv = lax.rsqrt(deg)
    hs_ref[...] = h * dinv
    dinv_ref[...] = dinv


def _mid_body(a0_ref, a1_ref, hs_ref, dinv_ref, b1_ref, w2_ref, hs2_ref):
    agg = a0_ref[...] + a1_ref[...] + hs_ref[...]
    dinv = dinv_ref[...]
    z = jnp.maximum(agg * dinv + b1_ref[...][None, :], 0.0)
    h2 = jnp.dot(z, w2_ref[...], preferred_element_type=jnp.float32)
    hs2_ref[...] = h2 * dinv


def _fin_body(a0_ref, a1_ref, hs2_ref, dinv_ref, b2_ref, out_ref, *, nvalid):
    agg = a0_ref[...] + a1_ref[...] + hs2_ref[...]
    logits = agg * dinv_ref[...] + b2_ref[...][None, :]
    col = lax.broadcasted_iota(jnp.int32, logits.shape, 1)
    valid = col < nvalid
    ml = jnp.where(valid, logits, jnp.float32(-1e30))
    m = jnp.max(ml, axis=1, keepdims=True)
    e = jnp.where(valid, jnp.exp(ml - m), 0.0)
    lse = jnp.log(jnp.sum(e, axis=1, keepdims=True)) + m
    out_ref[...] = (logits - lse)[:, :nvalid]


def kernel(x, edge_index, W1, b1, W2, b2):
    N, F = x.shape
    H = W1.shape[1]
    O = W2.shape[1]
    E = edge_index.shape[1]
    O8 = 16

    grain = NW * CH * KJ
    Ep = ((E + grain - 1) // grain) * grain
    pad = Ep - E
    Np = ((N + NW + 1023) // 1024) * 1024

    src = edge_index[0]
    dst = edge_index[1]
    src_p = jnp.concatenate(
        [src, jnp.zeros((pad,), jnp.int32)]).reshape(Ep // CH, CH)
    dst_p = jnp.concatenate(
        [dst, N + (jnp.arange(pad, dtype=jnp.int32) % NW)]).reshape(Ep // CH, CH)

    deg0, deg1 = _make_deg_kernel(Ep, Np)(dst_p)
    d0 = deg0.reshape(Np, 1)
    d1 = deg1.reshape(Np, 1)

    nb = N // BN
    hs, dinv = pl.pallas_call(
        _mm_body,
        grid=(nb,),
        in_specs=[
            pl.BlockSpec((BN, F), lambda i: (i, 0)),
            pl.BlockSpec((F, H), lambda i: (0, 0)),
            pl.BlockSpec((BN, 1), lambda i: (i, 0)),
            pl.BlockSpec((BN, 1), lambda i: (i, 0)),
        ],
        out_specs=[
            pl.BlockSpec((BN, H), lambda i: (i, 0)),
            pl.BlockSpec((BN, 1), lambda i: (i, 0)),
        ],
        out_shape=[
            jax.ShapeDtypeStruct((N, H), jnp.float32),
            jax.ShapeDtypeStruct((N, 1), jnp.float32),
        ],
    )(x, W1, d0, d1)

    a10, a11 = _make_scatter_kernel(H, Ep, Np, "gcn_scatter1")(
        src_p, dst_p, hs)

    W2p = jnp.pad(W2, ((0, 0), (0, O8 - O)))
    hs2 = pl.pallas_call(
        _mid_body,
        grid=(nb,),
        in_specs=[
            pl.BlockSpec((BN, H), lambda i: (i, 0)),
            pl.BlockSpec((BN, H), lambda i: (i, 0)),
            pl.BlockSpec((BN, H), lambda i: (i, 0)),
            pl.BlockSpec((BN, 1), lambda i: (i, 0)),
            pl.BlockSpec((H,), lambda i: (0,)),
            pl.BlockSpec((H, O8), lambda i: (0, 0)),
        ],
        out_specs=pl.BlockSpec((BN, O8), lambda i: (i, 0)),
        out_shape=jax.ShapeDtypeStruct((N, O8), jnp.float32),
    )(a10, a11, hs, dinv, b1, W2p)

    a20, a21 = _make_scatter_kernel(O8, Ep, Np, "gcn_scatter2")(
        src_p, dst_p, hs2)

    b2p = jnp.pad(b2, (0, O8 - O))
    out = pl.pallas_call(
        functools.partial(_fin_body, nvalid=O),
        grid=(nb,),
        in_specs=[
            pl.BlockSpec((BN, O8), lambda i: (i, 0)),
            pl.BlockSpec((BN, O8), lambda i: (i, 0)),
            pl.BlockSpec((BN, O8), lambda i: (i, 0)),
            pl.BlockSpec((BN, 1), lambda i: (i, 0)),
            pl.BlockSpec((O8,), lambda i: (0,)),
        ],
        out_specs=pl.BlockSpec((BN, O), lambda i: (i, 0)),
        out_shape=jax.ShapeDtypeStruct((N, O), jnp.float32),
    )(a20, a21, hs2, dinv, b2p)
    return out
